# baseline (speedup 1.0000x reference)
import functools
import math

import jax
import jax.numpy as jnp
from jax import lax
from jax.experimental import pallas as pl
from jax.experimental.pallas import tpu as pltpu
from jax.experimental.pallas import tpu_sc as plsc

NUM_HIDDENS = 64
SCALE = math.sqrt(NUM_HIDDENS)

_info = plsc.get_sparse_core_info()
NC, NS, L = _info.num_cores, _info.num_subcores, _info.num_lanes
NW = NC * NS

CHUNK = 640


def _make_kernel(B, D):
    assert B % NW == 0
    b_per_w = B // NW
    assert b_per_w % CHUNK == 0
    nchunks = b_per_w // CHUNK
    mesh = plsc.VectorSubcoreMesh(core_axis_name="c", subcore_axis_name="s")

    @functools.partial(
        pl.kernel,
        mesh=mesh,
        out_type=jax.ShapeDtypeStruct((B, D), jnp.float32),
        compiler_params=pltpu.CompilerParams(use_tc_tiling_on_sc=False),
        scratch_types=[
            pltpu.VMEM((b_per_w,), jnp.int32),
            pltpu.VMEM((CHUNK, D), jnp.float32),
            pltpu.VMEM((CHUNK, D), jnp.float32),
            pltpu.SemaphoreType.DMA,
            pltpu.SemaphoreType.DMA,
        ],
    )
    def emb(x_hbm, table_hbm, out_hbm, idx_v, buf0, buf1, sem0, sem1):
        wid = lax.axis_index("s") * NC + lax.axis_index("c")
        base = wid * b_per_w
        bufs = (buf0, buf1)
        sems = (sem0, sem1)

        pltpu.sync_copy(x_hbm.at[pl.ds(base, b_per_w)], idx_v)

        def gather(c):
            buf, sem = bufs[c % 2], sems[c % 2]
            idx = idx_v.at[pl.ds(c * CHUNK, CHUNK)]
            return pltpu.async_copy(table_hbm.at[idx], buf, sem)

        UNROLL = 8

        def scale_body(buf, i, _):
            for u in range(UNROLL):
                r = i * UNROLL + u
                for j in range(D // L):
                    sl = pl.ds(j * L, L)
                    buf[r, sl] = buf[r, sl] * SCALE
            return 0

        handle = gather(0)
        for c in range(nchunks):
            nxt = gather(c + 1) if c + 1 < nchunks else None
            handle.wait()
            buf = bufs[c % 2]
            lax.fori_loop(0, CHUNK // UNROLL, functools.partial(scale_body, buf), 0)
            pltpu.sync_copy(buf, out_hbm.at[pl.ds(base + c * CHUNK, CHUNK)])
            handle = nxt

    return emb


@jax.jit
def kernel(x, table):
    B = x.shape[0] * x.shape[1]
    D = table.shape[1]
    x_flat = x.reshape(B).astype(jnp.int32)
    out = _make_kernel(B, D)(x_flat, table)
    return out.reshape(x.shape[0], x.shape[1], D)
